# baseline (speedup 1.0000x reference)
import jax
import jax.numpy as jnp
from jax.experimental import pallas as pl
from jax.experimental.pallas import tpu as pltpu


def _fused_kernel(x_ref, adjd_ref, w_ref, out_ref,
                  adjn_sc, m0_acc, cs_acc, mn_sc):
    p = pl.program_id(0)
    i = pl.program_id(1)
    num_tiles = pl.num_programs(1)
    tile, lanes = adjd_ref.shape
    a = lanes // 2
    d_in = x_ref.shape[1]
    d_out = w_ref.shape[1]

    @pl.when(jnp.logical_and(p == 0, i == 0))
    def _init():
        m0_acc[...] = jnp.zeros_like(m0_acc)
        cs_acc[...] = jnp.zeros_like(cs_acc)

    @pl.when(p == 0)
    def _phase0():
        adjd = adjd_ref[...]
        x_bf = x_ref[...].astype(jnp.bfloat16)
        m0_acc[...] += jax.lax.dot_general(
            adjd, x_bf, (((0,), (0,)), ((), ())),
            preferred_element_type=jnp.float32)
        ones_rows = jnp.ones((tile, 8), dtype=jnp.bfloat16)
        cs_acc[...] += jax.lax.dot_general(
            ones_rows, adjd, (((0,), (0,)), ((), ())),
            preferred_element_type=jnp.float32)
        r_id = jax.lax.broadcasted_iota(jnp.int32, (lanes, 8), 0)
        half_ones = (r_id < a).astype(jnp.bfloat16)
        rsb = jax.lax.dot_general(
            adjd, half_ones, (((1,), (0,)), ((), ())),
            preferred_element_type=jnp.float32)
        rc = 1.0 / (rsb + 1e-12)
        adjn_sc[pl.ds(i * tile, tile), :] = (adjd * rc[:, 0:1]).astype(jnp.bfloat16)

        @pl.when(i == num_tiles - 1)
        def _finish():
            rcol = 1.0 / (cs_acc[0:1, :a] + 1e-12)
            row_id = jax.lax.broadcasted_iota(jnp.int32, (lanes, a), 0)
            col_id = jax.lax.broadcasted_iota(jnp.int32, (lanes, a), 1)
            dmt = jnp.where(row_id == col_id, rcol, 0.0)
            m0n = jax.lax.dot_general(
                dmt.astype(jnp.bfloat16), m0_acc[:a, :].astype(jnp.bfloat16),
                (((1,), (0,)), ((), ())), preferred_element_type=jnp.float32)
            mn = jax.lax.dot_general(
                m0n.astype(jnp.bfloat16), w_ref[...],
                (((1,), (0,)), ((), ())), preferred_element_type=jnp.float32)
            mn_sc[...] = mn.astype(jnp.bfloat16)

    @pl.when(p == 1)
    def _phase1():
        adjn = adjn_sc[pl.ds(i * tile, tile), :]
        out_ref[...] = jax.lax.dot_general(
            adjn, mn_sc[...], (((1,), (0,)), ((), ())),
            preferred_element_type=jnp.float32)


def _pick_tile(n):
    for t in (10000, 5000, 4000, 2500, 2000, 1000, 500, 200, 100, 40, 8):
        if n % t == 0 and t % 8 == 0:
            return t
    return n


def kernel(input, adj, W, anchor_mp):
    n, d_in = input.shape
    a = adj.shape[1]
    d_out = W.shape[1]
    tile = _pick_tile(n)
    num_tiles = n // tile

    eye = jnp.eye(a, dtype=jnp.bfloat16)
    gmat = jnp.concatenate([eye, eye], axis=1)
    adj_d = jax.lax.dot_general(
        adj.astype(jnp.bfloat16), gmat, (((1,), (0,)), ((), ())),
        preferred_element_type=jnp.bfloat16)
    w_scaled = (W * jnp.asarray(anchor_mp, W.dtype)).astype(jnp.bfloat16)

    out = pl.pallas_call(
        _fused_kernel,
        grid=(2, num_tiles),
        in_specs=[
            pl.BlockSpec((tile, d_in), lambda p, i: (i * (1 - p), 0)),
            pl.BlockSpec((tile, 2 * a), lambda p, i: (i * (1 - p), 0)),
            pl.BlockSpec((d_in, d_out), lambda p, i: (0, 0)),
        ],
        out_specs=pl.BlockSpec((tile, d_out), lambda p, i: (i * p, 0)),
        out_shape=jax.ShapeDtypeStruct((n, d_out), jnp.float32),
        scratch_shapes=[
            pltpu.VMEM((n, 2 * a), jnp.bfloat16),
            pltpu.VMEM((2 * a, d_in), jnp.float32),
            pltpu.VMEM((8, 2 * a), jnp.float32),
            pltpu.VMEM((2 * a, d_out), jnp.bfloat16),
        ],
    )(input, adj_d, w_scaled)
    return out


# speedup vs baseline: 1.0198x; 1.0198x over previous
import jax
import jax.numpy as jnp
from jax.experimental import pallas as pl
from jax.experimental.pallas import tpu as pltpu


def _fused_kernel(x_ref, adjd_ref, w_ref, out_ref,
                  adjn_sc, m0_acc, cs_acc, mn_sc):
    p = pl.program_id(0)
    i = pl.program_id(1)
    num_tiles = pl.num_programs(1)
    tile, lanes = adjd_ref.shape
    a = lanes // 2
    d_in = x_ref.shape[1]
    d_out = w_ref.shape[1]

    @pl.when(jnp.logical_and(p == 0, i == 0))
    def _init():
        m0_acc[...] = jnp.zeros_like(m0_acc)
        cs_acc[...] = jnp.zeros_like(cs_acc)

    @pl.when(p == 0)
    def _phase0():
        adjd = adjd_ref[...]
        x_bf = x_ref[...].astype(jnp.bfloat16)
        m0_acc[...] += jax.lax.dot_general(
            adjd, x_bf, (((0,), (0,)), ((), ())),
            preferred_element_type=jnp.float32)
        adjf = adjd.astype(jnp.float32)
        cs_acc[...] += jnp.sum(adjf, axis=0, keepdims=True)
        rs = jnp.sum(adjf[:, :a], axis=1, keepdims=True)
        rc = 1.0 / (rs + 1e-12)
        adjn_sc[pl.ds(i * tile, tile), :] = (adjd * rc).astype(jnp.bfloat16)

        @pl.when(i == num_tiles - 1)
        def _finish():
            rcol = 1.0 / (cs_acc[:, :a] + 1e-12)
            row_id = jax.lax.broadcasted_iota(jnp.int32, (lanes, a), 0)
            col_id = jax.lax.broadcasted_iota(jnp.int32, (lanes, a), 1)
            dmt = jnp.where(row_id == col_id, rcol, 0.0)
            m0n = jax.lax.dot_general(
                dmt.astype(jnp.bfloat16), m0_acc[:a, :].astype(jnp.bfloat16),
                (((1,), (0,)), ((), ())), preferred_element_type=jnp.float32)
            mn = jax.lax.dot_general(
                m0n.astype(jnp.bfloat16), w_ref[...],
                (((1,), (0,)), ((), ())), preferred_element_type=jnp.float32)
            mn_sc[...] = mn.astype(jnp.bfloat16)

    @pl.when(p == 1)
    def _phase1():
        adjn = adjn_sc[pl.ds(i * tile, tile), :]
        out_ref[...] = jax.lax.dot_general(
            adjn, mn_sc[...], (((1,), (0,)), ((), ())),
            preferred_element_type=jnp.float32)


def _pick_tile(n):
    for t in (10000, 5000, 4000, 2500, 2000, 1000, 500, 200, 100, 40, 8):
        if n % t == 0 and t % 8 == 0:
            return t
    return n


def kernel(input, adj, W, anchor_mp):
    n, d_in = input.shape
    a = adj.shape[1]
    d_out = W.shape[1]
    tile = _pick_tile(n)
    num_tiles = n // tile

    eye = jnp.eye(a, dtype=jnp.bfloat16)
    gmat = jnp.concatenate([eye, eye], axis=1)
    adj_d = jax.lax.dot_general(
        adj.astype(jnp.bfloat16), gmat, (((1,), (0,)), ((), ())),
        preferred_element_type=jnp.bfloat16)
    w_scaled = (W * jnp.asarray(anchor_mp, W.dtype)).astype(jnp.bfloat16)

    out = pl.pallas_call(
        _fused_kernel,
        grid=(2, num_tiles),
        in_specs=[
            pl.BlockSpec((tile, d_in), lambda p, i: (i * (1 - p), 0)),
            pl.BlockSpec((tile, 2 * a), lambda p, i: (i * (1 - p), 0)),
            pl.BlockSpec((d_in, d_out), lambda p, i: (0, 0)),
        ],
        out_specs=pl.BlockSpec((tile, d_out), lambda p, i: (i * p, 0)),
        out_shape=jax.ShapeDtypeStruct((n, d_out), jnp.float32),
        scratch_shapes=[
            pltpu.VMEM((n, 2 * a), jnp.bfloat16),
            pltpu.VMEM((2 * a, d_in), jnp.float32),
            pltpu.VMEM((1, 2 * a), jnp.float32),
            pltpu.VMEM((2 * a, d_out), jnp.bfloat16),
        ],
    )(input, adj_d, w_scaled)
    return out
